# baseline (speedup 1.0000x reference)
import functools

import jax
import jax.numpy as jnp
from jax import lax
from jax.experimental import pallas as pl
from jax.experimental.pallas import tpu as pltpu
from jax.experimental.pallas import tpu_sc as plsc

NNODE = 10000
DIM = 128
NGRAPH = 64
NC = 2
NS = 16
LANES = 16
NW = NC * NS
NPAD = 10240
RPT = NPAD // NS
EC = 128
WIN = 8
XCH = 5
XCS = 64
BN = 1024
GRID = NPAD // BN

_mesh = plsc.VectorSubcoreMesh(core_axis_name="c", subcore_axis_name="s")


def _sc_deg_emb(dst_idx, x_idx, emb, ew):
    @functools.partial(
        pl.kernel,
        out_type=[
            jax.ShapeDtypeStruct((NC, NPAD), jnp.float32),
            jax.ShapeDtypeStruct((NPAD, DIM), jnp.float32),
        ],
        mesh=_mesh,
        scratch_types=[
            pltpu.VMEM((ew, EC), jnp.int32),
            pltpu.VMEM((XCH, XCS), jnp.int32),
            pltpu.VMEM((XCS, DIM), jnp.float32),
            pltpu.VMEM((EC,), jnp.float32),
            pltpu.VMEM((RPT,), jnp.float32),
            pltpu.VMEM_SHARED((NPAD,), jnp.float32),
            pltpu.SemaphoreType.DMA,
        ],
    )
    def k(dst_hbm, x_hbm, emb_hbm, deg_out, h0_out,
          dst_v, x_v, rows_v, ones_v, z_v, deg_sh, sem):
        cid = lax.axis_index("c")
        sid = lax.axis_index("s")
        widx = cid * NS + sid
        for t in range(RPT // LANES):
            z_v[pl.ds(t * LANES, LANES)] = jnp.zeros((LANES,), jnp.float32)
        for t in range(EC // LANES):
            ones_v[pl.ds(t * LANES, LANES)] = jnp.ones((LANES,), jnp.float32)
        pltpu.sync_copy(z_v, deg_sh.at[pl.ds(sid * RPT, RPT)])
        plsc.subcore_barrier()
        pltpu.sync_copy(dst_hbm.at[widx, pl.ds(0, ew)], dst_v)
        pltpu.sync_copy(x_hbm.at[widx], x_v)
        for j in range(XCH):
            pltpu.async_copy(emb_hbm.at[x_v.at[j]], rows_v, sem).wait()
            pltpu.sync_copy(rows_v, h0_out.at[pl.ds(widx * XCH * XCS + j * XCS, XCS)])

        def body(j, carry):
            pltpu.sync_copy(ones_v, deg_sh.at[dst_v.at[j]], add=True)
            return carry

        lax.fori_loop(0, ew, body, 0)
        plsc.subcore_barrier()
        pltpu.sync_copy(deg_sh.at[pl.ds(sid * RPT, RPT)],
                        deg_out.at[cid, pl.ds(sid * RPT, RPT)])

    return k(dst_idx, x_idx, emb)


def _sc_edge(src_idx, dst_idx, g):
    ewt = src_idx.shape[1]
    ew = ewt - 2 * WIN
    npair = ew // (2 * WIN)

    @functools.partial(
        pl.kernel,
        out_type=jax.ShapeDtypeStruct((NC, NPAD, DIM), jnp.float32),
        mesh=_mesh,
        scratch_types=[
            pltpu.VMEM((WIN, EC), jnp.int32),
            pltpu.VMEM((WIN, EC), jnp.int32),
            pltpu.VMEM((WIN, EC), jnp.int32),
            pltpu.VMEM((WIN, EC), jnp.int32),
            pltpu.VMEM((EC, DIM), jnp.float32),
            pltpu.VMEM((EC, DIM), jnp.float32),
            pltpu.VMEM((LANES, DIM), jnp.float32),
            pltpu.VMEM_SHARED((NPAD, DIM), jnp.float32),
            pltpu.SemaphoreType.DMA,
            pltpu.SemaphoreType.DMA,
            pltpu.SemaphoreType.DMA,
        ],
    )
    def k(src_hbm, dst_hbm, g_hbm, acc_out,
          sA, dA, sB, dB, buf0, buf1, z_v, acc_sh, gsem0, gsem1, psem):
        cid = lax.axis_index("c")
        sid = lax.axis_index("s")
        widx = cid * NS + sid
        for r in range(LANES):
            for t in range(DIM // LANES):
                z_v[r, pl.ds(t * LANES, LANES)] = jnp.zeros((LANES,), jnp.float32)

        def zbody(t, carry):
            pltpu.sync_copy(z_v, acc_sh.at[pl.ds(sid * RPT + t * LANES, LANES)])
            return carry

        lax.fori_loop(0, RPT // LANES, zbody, 0)
        plsc.subcore_barrier()

        def prefetch(w, sw, dw):
            pltpu.async_copy(src_hbm.at[widx, pl.ds(w * WIN, WIN)], sw, psem)
            pltpu.async_copy(dst_hbm.at[widx, pl.ds(w * WIN, WIN)], dw, psem)

        def wait_prefetch(w, sw, dw):
            pltpu.make_async_copy(
                src_hbm.at[widx, pl.ds(w * WIN, WIN)], sw, psem).wait()
            pltpu.make_async_copy(
                dst_hbm.at[widx, pl.ds(w * WIN, WIN)], dw, psem).wait()

        def win_body(s_cur, d_cur, s_nxt, drain_next):
            for r in range(0, WIN, 2):
                pltpu.async_copy(g_hbm.at[s_cur.at[r + 1]], buf1, gsem1)
                pltpu.make_async_copy(g_hbm.at[s_cur.at[r]], buf0, gsem0).wait()
                pltpu.sync_copy(buf0, acc_sh.at[d_cur.at[r]], add=True)
                if r + 2 < WIN:
                    pltpu.async_copy(g_hbm.at[s_cur.at[r + 2]], buf0, gsem0)
                else:
                    drain_next()
                    pltpu.async_copy(g_hbm.at[s_nxt.at[0]], buf0, gsem0)
                pltpu.make_async_copy(g_hbm.at[s_cur.at[r + 1]], buf1, gsem1).wait()
                pltpu.sync_copy(buf1, acc_sh.at[d_cur.at[r + 1]], add=True)

        pltpu.sync_copy(src_hbm.at[widx, pl.ds(0, WIN)], sA)
        pltpu.sync_copy(dst_hbm.at[widx, pl.ds(0, WIN)], dA)
        pltpu.async_copy(g_hbm.at[sA.at[0]], buf0, gsem0)
        prefetch(1, sB, dB)

        def body(kk, carry):
            w0 = 2 * kk
            win_body(sA, dA, sB, lambda: wait_prefetch(w0 + 1, sB, dB))
            prefetch(w0 + 2, sA, dA)
            win_body(sB, dB, sA, lambda: wait_prefetch(w0 + 2, sA, dA))
            prefetch(w0 + 3, sB, dB)
            return carry

        lax.fori_loop(0, npair, body, 0)
        pltpu.make_async_copy(g_hbm.at[sA.at[0]], buf0, gsem0).wait()
        wait_prefetch(2 * npair + 1, sB, dB)
        plsc.subcore_barrier()
        pltpu.sync_copy(acc_sh.at[pl.ds(sid * RPT, RPT)],
                        acc_out.at[cid, pl.ds(sid * RPT, RPT)])

    return k(src_idx, dst_idx, g)


def _tc_pre(deg2, h0, w0):
    def body(deg_ref, h_ref, w_ref, g_ref, dinv_ref):
        d = deg_ref[0] + deg_ref[1] + 1.0
        dinv = lax.rsqrt(d)
        dinv_ref[...] = dinv
        g_ref[...] = dinv * jnp.dot(h_ref[...], w_ref[...],
                                    preferred_element_type=jnp.float32)

    return pl.pallas_call(
        body,
        grid=(GRID,),
        in_specs=[
            pl.BlockSpec((NC, BN, 1), lambda i: (0, i, 0)),
            pl.BlockSpec((BN, DIM), lambda i: (i, 0)),
            pl.BlockSpec((DIM, DIM), lambda i: (0, 0)),
        ],
        out_specs=[
            pl.BlockSpec((BN, DIM), lambda i: (i, 0)),
            pl.BlockSpec((BN, 1), lambda i: (i, 0)),
        ],
        out_shape=[
            jax.ShapeDtypeStruct((NPAD, DIM), jnp.float32),
            jax.ShapeDtypeStruct((NPAD, 1), jnp.float32),
        ],
    )(deg2, h0, w0)


def _tc_layer(acc, g, dinv, b, w_next):
    def body(acc_ref, g_ref, dinv_ref, b_ref, w_ref, out_ref):
        dinv = dinv_ref[...]
        h = dinv * (acc_ref[0] + acc_ref[1] + g_ref[...]) + b_ref[...]
        h = jnp.maximum(h, 0.0)
        out_ref[...] = dinv * jnp.dot(h, w_ref[...],
                                      preferred_element_type=jnp.float32)

    return pl.pallas_call(
        body,
        grid=(GRID,),
        in_specs=[
            pl.BlockSpec((NC, BN, DIM), lambda i: (0, i, 0)),
            pl.BlockSpec((BN, DIM), lambda i: (i, 0)),
            pl.BlockSpec((BN, 1), lambda i: (i, 0)),
            pl.BlockSpec((1, DIM), lambda i: (0, 0)),
            pl.BlockSpec((DIM, DIM), lambda i: (0, 0)),
        ],
        out_specs=pl.BlockSpec((BN, DIM), lambda i: (i, 0)),
        out_shape=jax.ShapeDtypeStruct((NPAD, DIM), jnp.float32),
    )(acc, g, dinv, b, w_next)


def _tc_final(acc, g, dinv, b, batch_row):
    def body(acc_ref, g_ref, dinv_ref, b_ref, bt_ref, out_ref, sum_ref, cnt_ref):
        i = pl.program_id(0)
        h = dinv_ref[...] * (acc_ref[0] + acc_ref[1] + g_ref[...]) + b_ref[...]
        ids = lax.broadcasted_iota(jnp.int32, (NGRAPH, BN), 0)
        oh = (ids == bt_ref[...]).astype(jnp.float32)
        psum = jnp.dot(oh, h, preferred_element_type=jnp.float32)
        pcnt = jnp.sum(oh, axis=1, keepdims=True)

        @pl.when(i == 0)
        def _():
            sum_ref[...] = psum
            cnt_ref[...] = pcnt

        @pl.when(i > 0)
        def _():
            sum_ref[...] += psum
            cnt_ref[...] += pcnt

        @pl.when(i == GRID - 1)
        def _():
            out_ref[...] = sum_ref[...] / jnp.maximum(cnt_ref[...], 1.0)

    return pl.pallas_call(
        body,
        grid=(GRID,),
        in_specs=[
            pl.BlockSpec((NC, BN, DIM), lambda i: (0, i, 0)),
            pl.BlockSpec((BN, DIM), lambda i: (i, 0)),
            pl.BlockSpec((BN, 1), lambda i: (i, 0)),
            pl.BlockSpec((1, DIM), lambda i: (0, 0)),
            pl.BlockSpec((1, BN), lambda i: (0, i)),
        ],
        out_specs=pl.BlockSpec((NGRAPH, DIM), lambda i: (0, 0)),
        out_shape=jax.ShapeDtypeStruct((NGRAPH, DIM), jnp.float32),
        scratch_shapes=[
            pltpu.VMEM((NGRAPH, DIM), jnp.float32),
            pltpu.VMEM((NGRAPH, 1), jnp.float32),
        ],
    )(acc, g, dinv, b, batch_row)


def kernel(x, edge_index, batch, emb, W0, W1, W2, W3, b0, b1, b2, b3):
    src = edge_index[0]
    dst = edge_index[1]
    e = src.shape[0]
    ew = -(-e // (NW * EC))
    ew = -(-ew // (2 * WIN)) * 2 * WIN
    epad = NW * EC * ew - e
    src_p = jnp.concatenate([src, jnp.zeros((epad,), jnp.int32)]).reshape(NW, ew, EC)
    dst_p = jnp.concatenate([dst, jnp.full((epad,), NPAD - 1, jnp.int32)]).reshape(NW, ew, EC)
    src_p = jnp.concatenate([src_p, jnp.zeros((NW, 2 * WIN, EC), jnp.int32)], axis=1)
    dst_p = jnp.concatenate([dst_p, jnp.full((NW, 2 * WIN, EC), NPAD - 1, jnp.int32)], axis=1)
    n = x.shape[0]
    x_p = jnp.concatenate([x, jnp.zeros((NPAD - n,), jnp.int32)]).reshape(NW, XCH, XCS)
    batch_p = jnp.concatenate(
        [batch.astype(jnp.int32), jnp.full((NPAD - n,), NGRAPH, jnp.int32)]
    ).reshape(1, NPAD)

    deg2, h0 = _sc_deg_emb(dst_p, x_p, emb, ew)
    g, dinv = _tc_pre(deg2.reshape(NC, NPAD, 1), h0, W0)
    for wn, bi in ((W1, b0), (W2, b1), (W3, b2)):
        acc = _sc_edge(src_p, dst_p, g)
        g = _tc_layer(acc, g, dinv, bi.reshape(1, DIM), wn)
    acc = _sc_edge(src_p, dst_p, g)
    return _tc_final(acc, g, dinv, b3.reshape(1, DIM), batch_p)


# speedup vs baseline: 1.1616x; 1.1616x over previous
import functools

import jax
import jax.numpy as jnp
from jax import lax
from jax.experimental import pallas as pl
from jax.experimental.pallas import tpu as pltpu
from jax.experimental.pallas import tpu_sc as plsc

NNODE = 10000
DIM = 128
NGRAPH = 64
NC = 2
NS = 16
LANES = 16
NW = NC * NS
NPAD = 10240
RPT = NPAD // NS
EC = 128
WIN = 8
XCH = 5
XCS = 64
BN = 1024
GRID = NPAD // BN

_mesh = plsc.VectorSubcoreMesh(core_axis_name="c", subcore_axis_name="s")


def _sc_deg_emb(dst_idx, x_idx, emb, ew):
    @functools.partial(
        pl.kernel,
        out_type=[
            jax.ShapeDtypeStruct((NC, NPAD), jnp.float32),
            jax.ShapeDtypeStruct((NPAD, DIM), jnp.float32),
        ],
        mesh=_mesh,
        scratch_types=[
            pltpu.VMEM((ew, EC), jnp.int32),
            pltpu.VMEM((XCH, XCS), jnp.int32),
            pltpu.VMEM((XCS, DIM), jnp.float32),
            pltpu.VMEM((EC,), jnp.float32),
            pltpu.VMEM((RPT,), jnp.float32),
            pltpu.VMEM_SHARED((NPAD,), jnp.float32),
            pltpu.SemaphoreType.DMA,
        ],
    )
    def k(dst_hbm, x_hbm, emb_hbm, deg_out, h0_out,
          dst_v, x_v, rows_v, ones_v, z_v, deg_sh, sem):
        cid = lax.axis_index("c")
        sid = lax.axis_index("s")
        widx = cid * NS + sid
        for t in range(RPT // LANES):
            z_v[pl.ds(t * LANES, LANES)] = jnp.zeros((LANES,), jnp.float32)
        for t in range(EC // LANES):
            ones_v[pl.ds(t * LANES, LANES)] = jnp.ones((LANES,), jnp.float32)
        pltpu.sync_copy(z_v, deg_sh.at[pl.ds(sid * RPT, RPT)])
        plsc.subcore_barrier()
        pltpu.sync_copy(dst_hbm.at[widx, pl.ds(0, ew)], dst_v)
        pltpu.sync_copy(x_hbm.at[widx], x_v)
        for j in range(XCH):
            pltpu.async_copy(emb_hbm.at[x_v.at[j]], rows_v, sem).wait()
            pltpu.sync_copy(rows_v, h0_out.at[pl.ds(widx * XCH * XCS + j * XCS, XCS)])

        def body(j, carry):
            pltpu.sync_copy(ones_v, deg_sh.at[dst_v.at[j]], add=True)
            return carry

        lax.fori_loop(0, ew, body, 0)
        plsc.subcore_barrier()
        pltpu.sync_copy(deg_sh.at[pl.ds(sid * RPT, RPT)],
                        deg_out.at[cid, pl.ds(sid * RPT, RPT)])

    return k(dst_idx, x_idx, emb)


def _sc_edge(src_idx, dst_idx, g):
    ewt = src_idx.shape[1]
    ew = ewt - 2 * WIN

    @functools.partial(
        pl.kernel,
        out_type=jax.ShapeDtypeStruct((NC, NPAD, DIM), jnp.float32),
        mesh=_mesh,
        scratch_types=[
            pltpu.VMEM((ew, EC), jnp.int32),
            pltpu.VMEM((ew, EC), jnp.int32),
            pltpu.VMEM((EC, DIM), jnp.float32),
            pltpu.VMEM((LANES, DIM), jnp.float32),
            pltpu.VMEM_SHARED((NPAD, DIM), jnp.float32),
            pltpu.SemaphoreType.DMA,
            pltpu.SemaphoreType.DMA,
        ],
    )
    def k(src_hbm, dst_hbm, g_hbm, acc_out,
          src_v, dst_v, buf0, z_v, acc_sh, gsem0, zsem):
        cid = lax.axis_index("c")
        sid = lax.axis_index("s")
        widx = cid * NS + sid
        for r in range(LANES):
            for t in range(DIM // LANES):
                z_v[r, pl.ds(t * LANES, LANES)] = jnp.zeros((LANES,), jnp.float32)

        for t in range(RPT // LANES):
            pltpu.async_copy(
                z_v, acc_sh.at[pl.ds(sid * RPT + t * LANES, LANES)], zsem)
        pltpu.async_copy(src_hbm.at[widx, pl.ds(0, ew)], src_v, gsem0)
        pltpu.async_copy(dst_hbm.at[widx, pl.ds(0, ew)], dst_v, gsem0)
        for t in range(RPT // LANES):
            pltpu.make_async_copy(
                z_v, acc_sh.at[pl.ds(sid * RPT + t * LANES, LANES)], zsem).wait()
        pltpu.make_async_copy(src_hbm.at[widx, pl.ds(0, ew)], src_v, gsem0).wait()
        pltpu.make_async_copy(dst_hbm.at[widx, pl.ds(0, ew)], dst_v, gsem0).wait()
        plsc.subcore_barrier()

        def body(j, carry):
            pltpu.async_copy(g_hbm.at[src_v.at[j]], buf0, gsem0).wait()
            pltpu.sync_copy(buf0, acc_sh.at[dst_v.at[j]], add=True)
            return carry

        lax.fori_loop(0, ew, body, 0)
        plsc.subcore_barrier()
        pltpu.sync_copy(acc_sh.at[pl.ds(sid * RPT, RPT)],
                        acc_out.at[cid, pl.ds(sid * RPT, RPT)])

    return k(src_idx, dst_idx, g)


def _tc_pre(deg2, h0, w0):
    def body(deg_ref, h_ref, w_ref, g_ref, dinv_ref):
        d = deg_ref[0] + deg_ref[1] + 1.0
        dinv = lax.rsqrt(d)
        dinv_ref[...] = dinv
        g_ref[...] = dinv * jnp.dot(h_ref[...], w_ref[...],
                                    preferred_element_type=jnp.float32)

    return pl.pallas_call(
        body,
        grid=(GRID,),
        in_specs=[
            pl.BlockSpec((NC, BN, 1), lambda i: (0, i, 0)),
            pl.BlockSpec((BN, DIM), lambda i: (i, 0)),
            pl.BlockSpec((DIM, DIM), lambda i: (0, 0)),
        ],
        out_specs=[
            pl.BlockSpec((BN, DIM), lambda i: (i, 0)),
            pl.BlockSpec((BN, 1), lambda i: (i, 0)),
        ],
        out_shape=[
            jax.ShapeDtypeStruct((NPAD, DIM), jnp.float32),
            jax.ShapeDtypeStruct((NPAD, 1), jnp.float32),
        ],
    )(deg2, h0, w0)


def _tc_layer(acc, g, dinv, b, w_next):
    def body(acc_ref, g_ref, dinv_ref, b_ref, w_ref, out_ref):
        dinv = dinv_ref[...]
        h = dinv * (acc_ref[0] + acc_ref[1] + g_ref[...]) + b_ref[...]
        h = jnp.maximum(h, 0.0)
        out_ref[...] = dinv * jnp.dot(h, w_ref[...],
                                      preferred_element_type=jnp.float32)

    return pl.pallas_call(
        body,
        grid=(GRID,),
        in_specs=[
            pl.BlockSpec((NC, BN, DIM), lambda i: (0, i, 0)),
            pl.BlockSpec((BN, DIM), lambda i: (i, 0)),
            pl.BlockSpec((BN, 1), lambda i: (i, 0)),
            pl.BlockSpec((1, DIM), lambda i: (0, 0)),
            pl.BlockSpec((DIM, DIM), lambda i: (0, 0)),
        ],
        out_specs=pl.BlockSpec((BN, DIM), lambda i: (i, 0)),
        out_shape=jax.ShapeDtypeStruct((NPAD, DIM), jnp.float32),
    )(acc, g, dinv, b, w_next)


def _tc_final(acc, g, dinv, b, batch_row):
    def body(acc_ref, g_ref, dinv_ref, b_ref, bt_ref, out_ref, sum_ref, cnt_ref):
        i = pl.program_id(0)
        h = dinv_ref[...] * (acc_ref[0] + acc_ref[1] + g_ref[...]) + b_ref[...]
        ids = lax.broadcasted_iota(jnp.int32, (NGRAPH, BN), 0)
        oh = (ids == bt_ref[...]).astype(jnp.float32)
        psum = jnp.dot(oh, h, preferred_element_type=jnp.float32)
        pcnt = jnp.sum(oh, axis=1, keepdims=True)

        @pl.when(i == 0)
        def _():
            sum_ref[...] = psum
            cnt_ref[...] = pcnt

        @pl.when(i > 0)
        def _():
            sum_ref[...] += psum
            cnt_ref[...] += pcnt

        @pl.when(i == GRID - 1)
        def _():
            out_ref[...] = sum_ref[...] / jnp.maximum(cnt_ref[...], 1.0)

    return pl.pallas_call(
        body,
        grid=(GRID,),
        in_specs=[
            pl.BlockSpec((NC, BN, DIM), lambda i: (0, i, 0)),
            pl.BlockSpec((BN, DIM), lambda i: (i, 0)),
            pl.BlockSpec((BN, 1), lambda i: (i, 0)),
            pl.BlockSpec((1, DIM), lambda i: (0, 0)),
            pl.BlockSpec((1, BN), lambda i: (0, i)),
        ],
        out_specs=pl.BlockSpec((NGRAPH, DIM), lambda i: (0, 0)),
        out_shape=jax.ShapeDtypeStruct((NGRAPH, DIM), jnp.float32),
        scratch_shapes=[
            pltpu.VMEM((NGRAPH, DIM), jnp.float32),
            pltpu.VMEM((NGRAPH, 1), jnp.float32),
        ],
    )(acc, g, dinv, b, batch_row)


def kernel(x, edge_index, batch, emb, W0, W1, W2, W3, b0, b1, b2, b3):
    src = edge_index[0]
    dst = edge_index[1]
    e = src.shape[0]
    ew = -(-e // (NW * EC))
    ew = -(-ew // (2 * WIN)) * 2 * WIN
    epad = NW * EC * ew - e
    src_p = jnp.concatenate([src, jnp.zeros((epad,), jnp.int32)]).reshape(NW, ew, EC)
    dst_p = jnp.concatenate([dst, jnp.full((epad,), NPAD - 1, jnp.int32)]).reshape(NW, ew, EC)
    src_p = jnp.concatenate([src_p, jnp.zeros((NW, 2 * WIN, EC), jnp.int32)], axis=1)
    dst_p = jnp.concatenate([dst_p, jnp.full((NW, 2 * WIN, EC), NPAD - 1, jnp.int32)], axis=1)
    n = x.shape[0]
    x_p = jnp.concatenate([x, jnp.zeros((NPAD - n,), jnp.int32)]).reshape(NW, XCH, XCS)
    batch_p = jnp.concatenate(
        [batch.astype(jnp.int32), jnp.full((NPAD - n,), NGRAPH, jnp.int32)]
    ).reshape(1, NPAD)

    deg2, h0 = _sc_deg_emb(dst_p, x_p, emb, ew)
    g, dinv = _tc_pre(deg2.reshape(NC, NPAD, 1), h0, W0)
    for wn, bi in ((W1, b0), (W2, b1), (W3, b2)):
        acc = _sc_edge(src_p, dst_p, g)
        g = _tc_layer(acc, g, dinv, bi.reshape(1, DIM), wn)
    acc = _sc_edge(src_p, dst_p, g)
    return _tc_final(acc, g, dinv, b3.reshape(1, DIM), batch_p)


# speedup vs baseline: 2.0051x; 1.7262x over previous
import functools

import jax
import jax.numpy as jnp
from jax import lax
from jax.experimental import pallas as pl
from jax.experimental.pallas import tpu as pltpu
from jax.experimental.pallas import tpu_sc as plsc

NNODE = 10000
DIM = 128
NGRAPH = 64
NC = 2
NS = 16
LANES = 16
NW = NC * NS
NPAD = 10240
RPT = NPAD // NS
EC = 128
WIN = 8
XCH = 5
XCS = 64
BN = 1024
GRID = NPAD // BN

_mesh = plsc.VectorSubcoreMesh(core_axis_name="c", subcore_axis_name="s")


def _sc_deg_emb(dst_idx, x_idx, emb):
    ew = dst_idx.shape[1]

    @functools.partial(
        pl.kernel,
        out_type=[
            jax.ShapeDtypeStruct((NC, NPAD), jnp.float32),
            jax.ShapeDtypeStruct((NPAD, DIM), jnp.float32),
        ],
        mesh=_mesh,
        scratch_types=[
            pltpu.VMEM((ew, EC), jnp.int32),
            pltpu.VMEM((XCH, XCS), jnp.int32),
            pltpu.VMEM((XCS, DIM), jnp.float32),
            pltpu.VMEM((EC,), jnp.float32),
            pltpu.VMEM((RPT,), jnp.float32),
            pltpu.VMEM_SHARED((NPAD,), jnp.float32),
            pltpu.SemaphoreType.DMA,
        ],
    )
    def k(dst_hbm, x_hbm, emb_hbm, deg_out, h0_out,
          dst_v, x_v, rows_v, ones_v, z_v, deg_sh, sem):
        cid = lax.axis_index("c")
        sid = lax.axis_index("s")
        widx = cid * NS + sid
        for t in range(RPT // LANES):
            z_v[pl.ds(t * LANES, LANES)] = jnp.zeros((LANES,), jnp.float32)
        for t in range(EC // LANES):
            ones_v[pl.ds(t * LANES, LANES)] = jnp.ones((LANES,), jnp.float32)
        pltpu.sync_copy(z_v, deg_sh.at[pl.ds(sid * RPT, RPT)])
        plsc.subcore_barrier()
        pltpu.sync_copy(dst_hbm.at[widx], dst_v)
        pltpu.sync_copy(x_hbm.at[widx], x_v)
        for j in range(XCH):
            pltpu.async_copy(emb_hbm.at[x_v.at[j]], rows_v, sem).wait()
            pltpu.sync_copy(rows_v, h0_out.at[pl.ds(widx * XCH * XCS + j * XCS, XCS)])

        def body(j, carry):
            pltpu.sync_copy(ones_v, deg_sh.at[dst_v.at[j]], add=True)
            return carry

        lax.fori_loop(0, ew, body, 0)
        plsc.subcore_barrier()
        pltpu.sync_copy(deg_sh.at[pl.ds(sid * RPT, RPT)],
                        deg_out.at[cid, pl.ds(sid * RPT, RPT)])

    return k(dst_idx, x_idx, emb)


def _sc_edge(src_idx, dst_idx, g):
    ew = src_idx.shape[1]

    @functools.partial(
        pl.kernel,
        out_type=jax.ShapeDtypeStruct((NC, NPAD, DIM), jnp.float32),
        mesh=_mesh,
        scratch_types=[
            pltpu.VMEM((ew, EC), jnp.int32),
            pltpu.VMEM((ew, EC), jnp.int32),
            pltpu.VMEM((EC, DIM), jnp.float32),
            pltpu.VMEM((LANES, DIM), jnp.float32),
            pltpu.VMEM_SHARED((NPAD, DIM), jnp.float32),
            pltpu.SemaphoreType.DMA,
        ],
    )
    def k(src_hbm, dst_hbm, g_hbm, acc_out,
          src_v, dst_v, rows_v, z_v, acc_sh, gsem):
        cid = lax.axis_index("c")
        sid = lax.axis_index("s")
        widx = cid * NS + sid
        for r in range(LANES):
            for t in range(DIM // LANES):
                z_v[r, pl.ds(t * LANES, LANES)] = jnp.zeros((LANES,), jnp.float32)

        def zbody(t, carry):
            pltpu.sync_copy(z_v, acc_sh.at[pl.ds(sid * RPT + t * LANES, LANES)])
            return carry

        lax.fori_loop(0, RPT // LANES, zbody, 0)
        plsc.subcore_barrier()
        pltpu.sync_copy(src_hbm.at[widx], src_v)
        pltpu.sync_copy(dst_hbm.at[widx], dst_v)

        def body(j, carry):
            pltpu.async_copy(g_hbm.at[src_v.at[j]], rows_v, gsem).wait()
            pltpu.sync_copy(rows_v, acc_sh.at[dst_v.at[j]], add=True)
            return carry

        lax.fori_loop(0, ew, body, 0)
        plsc.subcore_barrier()
        pltpu.sync_copy(acc_sh.at[pl.ds(sid * RPT, RPT)],
                        acc_out.at[cid, pl.ds(sid * RPT, RPT)])

    return k(src_idx, dst_idx, g)


def _tc_pre(deg2, h0, w0):
    def body(deg_ref, h_ref, w_ref, g_ref, dinv_ref):
        d = deg_ref[0] + deg_ref[1] + 1.0
        dinv = lax.rsqrt(d)
        dinv_ref[...] = dinv
        g_ref[...] = dinv * jnp.dot(h_ref[...], w_ref[...],
                                    preferred_element_type=jnp.float32)

    return pl.pallas_call(
        body,
        grid=(GRID,),
        in_specs=[
            pl.BlockSpec((NC, BN, 1), lambda i: (0, i, 0)),
            pl.BlockSpec((BN, DIM), lambda i: (i, 0)),
            pl.BlockSpec((DIM, DIM), lambda i: (0, 0)),
        ],
        out_specs=[
            pl.BlockSpec((BN, DIM), lambda i: (i, 0)),
            pl.BlockSpec((BN, 1), lambda i: (i, 0)),
        ],
        out_shape=[
            jax.ShapeDtypeStruct((NPAD, DIM), jnp.float32),
            jax.ShapeDtypeStruct((NPAD, 1), jnp.float32),
        ],
    )(deg2, h0, w0)


def _tc_layer(acc, g, dinv, b, w_next):
    def body(acc_ref, g_ref, dinv_ref, b_ref, w_ref, out_ref):
        dinv = dinv_ref[...]
        h = dinv * (acc_ref[0] + acc_ref[1] + g_ref[...]) + b_ref[...]
        h = jnp.maximum(h, 0.0)
        out_ref[...] = dinv * jnp.dot(h, w_ref[...],
                                      preferred_element_type=jnp.float32)

    return pl.pallas_call(
        body,
        grid=(GRID,),
        in_specs=[
            pl.BlockSpec((NC, BN, DIM), lambda i: (0, i, 0)),
            pl.BlockSpec((BN, DIM), lambda i: (i, 0)),
            pl.BlockSpec((BN, 1), lambda i: (i, 0)),
            pl.BlockSpec((1, DIM), lambda i: (0, 0)),
            pl.BlockSpec((DIM, DIM), lambda i: (0, 0)),
        ],
        out_specs=pl.BlockSpec((BN, DIM), lambda i: (i, 0)),
        out_shape=jax.ShapeDtypeStruct((NPAD, DIM), jnp.float32),
    )(acc, g, dinv, b, w_next)


def _tc_final(acc, g, dinv, b, batch_row):
    def body(acc_ref, g_ref, dinv_ref, b_ref, bt_ref, out_ref, sum_ref, cnt_ref):
        i = pl.program_id(0)
        h = dinv_ref[...] * (acc_ref[0] + acc_ref[1] + g_ref[...]) + b_ref[...]
        ids = lax.broadcasted_iota(jnp.int32, (NGRAPH, BN), 0)
        oh = (ids == bt_ref[...]).astype(jnp.float32)
        psum = jnp.dot(oh, h, preferred_element_type=jnp.float32)
        pcnt = jnp.sum(oh, axis=1, keepdims=True)

        @pl.when(i == 0)
        def _():
            sum_ref[...] = psum
            cnt_ref[...] = pcnt

        @pl.when(i > 0)
        def _():
            sum_ref[...] += psum
            cnt_ref[...] += pcnt

        @pl.when(i == GRID - 1)
        def _():
            out_ref[...] = sum_ref[...] / jnp.maximum(cnt_ref[...], 1.0)

    return pl.pallas_call(
        body,
        grid=(GRID,),
        in_specs=[
            pl.BlockSpec((NC, BN, DIM), lambda i: (0, i, 0)),
            pl.BlockSpec((BN, DIM), lambda i: (i, 0)),
            pl.BlockSpec((BN, 1), lambda i: (i, 0)),
            pl.BlockSpec((1, DIM), lambda i: (0, 0)),
            pl.BlockSpec((1, BN), lambda i: (0, i)),
        ],
        out_specs=pl.BlockSpec((NGRAPH, DIM), lambda i: (0, 0)),
        out_shape=jax.ShapeDtypeStruct((NGRAPH, DIM), jnp.float32),
        scratch_shapes=[
            pltpu.VMEM((NGRAPH, DIM), jnp.float32),
            pltpu.VMEM((NGRAPH, 1), jnp.float32),
        ],
    )(acc, g, dinv, b, batch_row)


def kernel(x, edge_index, batch, emb, W0, W1, W2, W3, b0, b1, b2, b3):
    src = edge_index[0]
    dst = edge_index[1]
    e = src.shape[0]
    ew = -(-e // (NW * EC))
    epad = NW * EC * ew - e
    src_p = jnp.concatenate([src, jnp.zeros((epad,), jnp.int32)]).reshape(NW, ew, EC)
    dst_p = jnp.concatenate([dst, jnp.full((epad,), NPAD - 1, jnp.int32)]).reshape(NW, ew, EC)
    n = x.shape[0]
    x_p = jnp.concatenate([x, jnp.zeros((NPAD - n,), jnp.int32)]).reshape(NW, XCH, XCS)
    batch_p = jnp.concatenate(
        [batch.astype(jnp.int32), jnp.full((NPAD - n,), NGRAPH, jnp.int32)]
    ).reshape(1, NPAD)

    deg2, h0 = _sc_deg_emb(dst_p, x_p, emb)
    g, dinv = _tc_pre(deg2.reshape(NC, NPAD, 1), h0, W0)
    for wn, bi in ((W1, b0), (W2, b1), (W3, b2)):
        acc = _sc_edge(src_p, dst_p, g)
        g = _tc_layer(acc, g, dinv, bi.reshape(1, DIM), wn)
    acc = _sc_edge(src_p, dst_p, g)
    return _tc_final(acc, g, dinv, b3.reshape(1, DIM), batch_p)


# speedup vs baseline: 2.6574x; 1.3253x over previous
import functools

import jax
import jax.numpy as jnp
from jax import lax
from jax.experimental import pallas as pl
from jax.experimental.pallas import tpu as pltpu
from jax.experimental.pallas import tpu_sc as plsc

NNODE = 10000
DIM = 128
NGRAPH = 64
NC = 2
NS = 16
LANES = 16
NW = NC * NS
NPAD = 10240
RPT = NPAD // NS
EC = 128
WIN = 8
XCH = 5
XCS = 64
BN = 1024
GRID = NPAD // BN

_mesh = plsc.VectorSubcoreMesh(core_axis_name="c", subcore_axis_name="s")


def _sc_deg_emb(dst_idx, x_idx, emb):
    ew = dst_idx.shape[1]

    @functools.partial(
        pl.kernel,
        out_type=[
            jax.ShapeDtypeStruct((NC, NPAD), jnp.float32),
            jax.ShapeDtypeStruct((NPAD, DIM), jnp.float32),
        ],
        mesh=_mesh,
        scratch_types=[
            pltpu.VMEM((ew, EC), jnp.int32),
            pltpu.VMEM((XCH, XCS), jnp.int32),
            pltpu.VMEM((XCS, DIM), jnp.float32),
            pltpu.VMEM((EC,), jnp.float32),
            pltpu.VMEM((RPT,), jnp.float32),
            pltpu.VMEM_SHARED((NPAD,), jnp.float32),
            pltpu.SemaphoreType.DMA,
        ],
    )
    def k(dst_hbm, x_hbm, emb_hbm, deg_out, h0_out,
          dst_v, x_v, rows_v, ones_v, z_v, deg_sh, sem):
        cid = lax.axis_index("c")
        sid = lax.axis_index("s")
        widx = cid * NS + sid
        for t in range(RPT // LANES):
            z_v[pl.ds(t * LANES, LANES)] = jnp.zeros((LANES,), jnp.float32)
        for t in range(EC // LANES):
            ones_v[pl.ds(t * LANES, LANES)] = jnp.ones((LANES,), jnp.float32)
        pltpu.sync_copy(z_v, deg_sh.at[pl.ds(sid * RPT, RPT)])
        plsc.subcore_barrier()
        pltpu.sync_copy(dst_hbm.at[widx], dst_v)
        pltpu.sync_copy(x_hbm.at[widx], x_v)
        for j in range(XCH):
            pltpu.async_copy(emb_hbm.at[x_v.at[j]], rows_v, sem).wait()
            pltpu.sync_copy(rows_v, h0_out.at[pl.ds(widx * XCH * XCS + j * XCS, XCS)])

        def body(j, carry):
            pltpu.sync_copy(ones_v, deg_sh.at[dst_v.at[j]], add=True)
            return carry

        lax.fori_loop(0, ew, body, 0)
        plsc.subcore_barrier()
        pltpu.sync_copy(deg_sh.at[pl.ds(sid * RPT, RPT)],
                        deg_out.at[cid, pl.ds(sid * RPT, RPT)])

    return k(dst_idx, x_idx, emb)


def _sc_edge(src_idx, dst_idx, g, ew0, ew1):
    ew = src_idx.shape[1]

    @functools.partial(
        pl.kernel,
        out_type=jax.ShapeDtypeStruct((NC, NPAD, DIM), jnp.float32),
        mesh=_mesh,
        scratch_types=[
            pltpu.VMEM((ew, EC), jnp.int32),
            pltpu.VMEM((ew, EC), jnp.int32),
            pltpu.VMEM((EC, DIM), jnp.float32),
            pltpu.VMEM((LANES, DIM), jnp.float32),
            pltpu.VMEM_SHARED((NPAD, DIM), jnp.float32),
            pltpu.SemaphoreType.DMA,
        ],
    )
    def k(src_hbm, dst_hbm, g_hbm, acc_out,
          src_v, dst_v, rows_v, z_v, acc_sh, gsem):
        cid = lax.axis_index("c")
        sid = lax.axis_index("s")
        widx = cid * NS + sid
        for r in range(LANES):
            for t in range(DIM // LANES):
                z_v[r, pl.ds(t * LANES, LANES)] = jnp.zeros((LANES,), jnp.float32)

        def zbody(t, carry):
            pltpu.sync_copy(z_v, acc_sh.at[pl.ds(sid * RPT + t * LANES, LANES)])
            return carry

        lax.fori_loop(0, RPT // LANES, zbody, 0)
        plsc.subcore_barrier()
        pltpu.sync_copy(src_hbm.at[widx], src_v)
        pltpu.sync_copy(dst_hbm.at[widx], dst_v)

        def body(j, carry):
            pltpu.async_copy(g_hbm.at[src_v.at[j]], rows_v, gsem).wait()
            pltpu.sync_copy(rows_v, acc_sh.at[dst_v.at[j]], add=True)
            return carry

        lax.fori_loop(0, jnp.where(cid == 0, ew0, ew1), body, 0)
        plsc.subcore_barrier()
        pltpu.sync_copy(acc_sh.at[pl.ds(sid * RPT, RPT)],
                        acc_out.at[cid, pl.ds(sid * RPT, RPT)])

    return k(src_idx, dst_idx, g)


def _tc_pre(deg2, h0, w0):
    def body(deg_ref, h_ref, w_ref, g_ref, dinv_ref):
        d = deg_ref[0] + deg_ref[1] + 1.0
        dinv = lax.rsqrt(d)
        dinv_ref[...] = dinv
        g_ref[...] = dinv * jnp.dot(h_ref[...], w_ref[...],
                                    preferred_element_type=jnp.float32)

    return pl.pallas_call(
        body,
        grid=(GRID,),
        in_specs=[
            pl.BlockSpec((NC, BN, 1), lambda i: (0, i, 0)),
            pl.BlockSpec((BN, DIM), lambda i: (i, 0)),
            pl.BlockSpec((DIM, DIM), lambda i: (0, 0)),
        ],
        out_specs=[
            pl.BlockSpec((BN, DIM), lambda i: (i, 0)),
            pl.BlockSpec((BN, 1), lambda i: (i, 0)),
        ],
        out_shape=[
            jax.ShapeDtypeStruct((NPAD, DIM), jnp.float32),
            jax.ShapeDtypeStruct((NPAD, 1), jnp.float32),
        ],
    )(deg2, h0, w0)


def _tc_layer(acc, g, dinv, b, w_next):
    def body(acc_ref, g_ref, dinv_ref, b_ref, w_ref, out_ref):
        dinv = dinv_ref[...]
        h = dinv * (acc_ref[0] + acc_ref[1] + g_ref[...]) + b_ref[...]
        h = jnp.maximum(h, 0.0)
        out_ref[...] = dinv * jnp.dot(h, w_ref[...],
                                      preferred_element_type=jnp.float32)

    return pl.pallas_call(
        body,
        grid=(GRID,),
        in_specs=[
            pl.BlockSpec((NC, BN, DIM), lambda i: (0, i, 0)),
            pl.BlockSpec((BN, DIM), lambda i: (i, 0)),
            pl.BlockSpec((BN, 1), lambda i: (i, 0)),
            pl.BlockSpec((1, DIM), lambda i: (0, 0)),
            pl.BlockSpec((DIM, DIM), lambda i: (0, 0)),
        ],
        out_specs=pl.BlockSpec((BN, DIM), lambda i: (i, 0)),
        out_shape=jax.ShapeDtypeStruct((NPAD, DIM), jnp.float32),
    )(acc, g, dinv, b, w_next)


def _tc_final(acc, g, dinv, b, batch_row):
    def body(acc_ref, g_ref, dinv_ref, b_ref, bt_ref, out_ref, sum_ref, cnt_ref):
        i = pl.program_id(0)
        h = dinv_ref[...] * (acc_ref[0] + acc_ref[1] + g_ref[...]) + b_ref[...]
        ids = lax.broadcasted_iota(jnp.int32, (NGRAPH, BN), 0)
        oh = (ids == bt_ref[...]).astype(jnp.float32)
        psum = jnp.dot(oh, h, preferred_element_type=jnp.float32)
        pcnt = jnp.sum(oh, axis=1, keepdims=True)

        @pl.when(i == 0)
        def _():
            sum_ref[...] = psum
            cnt_ref[...] = pcnt

        @pl.when(i > 0)
        def _():
            sum_ref[...] += psum
            cnt_ref[...] += pcnt

        @pl.when(i == GRID - 1)
        def _():
            out_ref[...] = sum_ref[...] / jnp.maximum(cnt_ref[...], 1.0)

    return pl.pallas_call(
        body,
        grid=(GRID,),
        in_specs=[
            pl.BlockSpec((NC, BN, DIM), lambda i: (0, i, 0)),
            pl.BlockSpec((BN, DIM), lambda i: (i, 0)),
            pl.BlockSpec((BN, 1), lambda i: (i, 0)),
            pl.BlockSpec((1, DIM), lambda i: (0, 0)),
            pl.BlockSpec((1, BN), lambda i: (0, i)),
        ],
        out_specs=pl.BlockSpec((NGRAPH, DIM), lambda i: (0, 0)),
        out_shape=jax.ShapeDtypeStruct((NGRAPH, DIM), jnp.float32),
        scratch_shapes=[
            pltpu.VMEM((NGRAPH, DIM), jnp.float32),
            pltpu.VMEM((NGRAPH, 1), jnp.float32),
        ],
    )(acc, g, dinv, b, batch_row)


def kernel(x, edge_index, batch, emb, W0, W1, W2, W3, b0, b1, b2, b3):
    src = edge_index[0]
    dst = edge_index[1]
    e = src.shape[0]
    ewt = -(-e // EC)
    ew0 = -(-(ewt * 3) // (5 * NS))
    e0 = NS * ew0 * EC
    ew1 = -(-(e - e0) // (NS * EC))
    ewm = max(ew0, ew1)

    def _part(arr, fill, rows):
        cap = NS * rows * EC
        a = jnp.concatenate(
            [arr, jnp.full((cap - arr.shape[0],), fill, jnp.int32)]
        ).reshape(NS, rows, EC)
        if rows < ewm:
            a = jnp.concatenate(
                [a, jnp.full((NS, ewm - rows, EC), fill, jnp.int32)], axis=1)
        return a

    src_p = jnp.concatenate([_part(src[:e0], 0, ew0),
                             _part(src[e0:], 0, ew1)], axis=0)
    dst_p = jnp.concatenate([_part(dst[:e0], NPAD - 1, ew0),
                             _part(dst[e0:], NPAD - 1, ew1)], axis=0)
    n = x.shape[0]
    x_p = jnp.concatenate([x, jnp.zeros((NPAD - n,), jnp.int32)]).reshape(NW, XCH, XCS)
    batch_p = jnp.concatenate(
        [batch.astype(jnp.int32), jnp.full((NPAD - n,), NGRAPH, jnp.int32)]
    ).reshape(1, NPAD)

    deg2, h0 = _sc_deg_emb(dst_p, x_p, emb)
    g, dinv = _tc_pre(deg2.reshape(NC, NPAD, 1), h0, W0)
    for wn, bi in ((W1, b0), (W2, b1), (W3, b2)):
        acc = _sc_edge(src_p, dst_p, g, ew0, ew1)
        g = _tc_layer(acc, g, dinv, bi.reshape(1, DIM), wn)
    acc = _sc_edge(src_p, dst_p, g, ew0, ew1)
    return _tc_final(acc, g, dinv, b3.reshape(1, DIM), batch_p)
